# Initial kernel scaffold; baseline (speedup 1.0000x reference)
import jax
import jax.numpy as jnp
from jax.experimental import pallas as pl


def kernel(x, table, W, b):
    raise NotImplementedError("write your pallas kernel here")


# speedup vs baseline: 6.0296x; 6.0296x over previous
import functools

import jax
import jax.numpy as jnp
from jax import lax
from jax.experimental import pallas as pl
from jax.experimental.pallas import tpu as pltpu
from jax.experimental.pallas import tpu_sc as plsc

TOK_DIM = 32
EMB_DIM = 128
BATCH = 4096
HIST = 50
N = BATCH * HIST

NC = 2
NS = 16
NW = NC * NS
G = 128
NG = N // G
GPW = NG // NW
K = 10
NSTEP = GPW // K


def _sc_gather(table, idx_grp):
    mesh = plsc.VectorSubcoreMesh(core_axis_name="c", subcore_axis_name="s")

    @functools.partial(
        pl.kernel,
        mesh=mesh,
        compiler_params=pltpu.CompilerParams(use_tc_tiling_on_sc=False),
        out_type=jax.ShapeDtypeStruct((N, TOK_DIM), jnp.float32),
        scratch_types=[
            pltpu.VMEM((GPW, G), jnp.int32),
            pltpu.VMEM((K * G, TOK_DIM), jnp.float32),
            pltpu.SemaphoreType.DMA,
        ],
    )
    def k(table_hbm, idx_hbm, out_hbm, idx_v, rows_v, sem):
        wid = lax.axis_index("s") * NC + lax.axis_index("c")
        rbase = wid * GPW * G
        pltpu.sync_copy(idx_hbm.at[wid], idx_v)
        for t in range(NSTEP):
            cps = []
            for j in range(K):
                cp = pltpu.async_copy(
                    table_hbm.at[idx_v.at[t * K + j]],
                    rows_v.at[pl.ds(j * G, G)],
                    sem,
                )
                cps.append(cp)
            for cp in cps:
                cp.wait()
            pltpu.sync_copy(rows_v, out_hbm.at[pl.ds(rbase + t * K * G, K * G)])

    return k(table, idx_grp)


def _proj_body(emb_ref, w_ref, b_ref, out_ref):
    out_ref[...] = (
        jnp.dot(emb_ref[...], w_ref[...], preferred_element_type=jnp.float32)
        + b_ref[...]
    )


def _tc_project(emb, wt, b2):
    R = 4096
    return pl.pallas_call(
        _proj_body,
        grid=(N // R,),
        in_specs=[
            pl.BlockSpec((R, TOK_DIM), lambda i: (i, 0)),
            pl.BlockSpec((TOK_DIM, EMB_DIM), lambda i: (0, 0)),
            pl.BlockSpec((1, EMB_DIM), lambda i: (0, 0)),
        ],
        out_specs=pl.BlockSpec((R, EMB_DIM), lambda i: (i, 0)),
        out_shape=jax.ShapeDtypeStruct((N, EMB_DIM), jnp.float32),
    )(emb, wt, b2)


def kernel(x, table, W, b):
    idx_grp = x.reshape(NW, GPW, G).astype(jnp.int32)
    emb = _sc_gather(table, idx_grp)
    out = _tc_project(emb, W.T, b.reshape(1, EMB_DIM))
    return out.reshape(BATCH, HIST, EMB_DIM)
